# baseline (speedup 1.0000x reference)
import jax
import jax.numpy as jnp
from jax.experimental import pallas as pl
from jax.experimental.pallas import tpu as pltpu

ADJ_POWERS = (0, 1, 2)
F_IN = 64
F_OUT = 32
NEG_SLOPE = 0.01

BN = 512
BM = 512
BR = 2048


def _leaky(v):
    return jnp.where(v >= 0, v, NEG_SLOPE * v)


def _prep_kernel(xt_ref, w_ref, b_ref, y0_ref, y1_ref, y2_ref):
    y = jnp.dot(xt_ref[0], w_ref[...], preferred_element_type=jnp.float32)
    y = y + b_ref[0][None, :]
    y0_ref[0] = _leaky(y[:, :F_OUT])
    y1_ref[0] = y[:, F_OUT:2 * F_OUT]
    y2_ref[0] = y[:, 2 * F_OUT:]


def _spmm2_kernel(adj_ref, g1_ref, g2_ref, h1_ref, u2_ref, acc1, acc2):
    j = pl.program_id(2)
    nj = pl.num_programs(2)
    a = adj_ref[0]
    g1 = g1_ref[0, pl.ds(j * BM, BM), :]
    g2 = g2_ref[0, pl.ds(j * BM, BM), :]
    p1 = jnp.dot(a, g1, preferred_element_type=jnp.float32)
    p2 = jnp.dot(a, g2, preferred_element_type=jnp.float32)

    @pl.when(j == 0)
    def _init():
        acc1[...] = p1
        acc2[...] = p2

    @pl.when(j > 0)
    def _accum():
        acc1[...] += p1
        acc2[...] += p2

    @pl.when(j == nj - 1)
    def _finish():
        h1_ref[0] = _leaky(acc1[...])
        u2_ref[0] = acc2[...]


def _spmm1_kernel(adj_ref, g_ref, h_ref, acc):
    j = pl.program_id(2)
    nj = pl.num_programs(2)
    a = adj_ref[0]
    g = g_ref[0, pl.ds(j * BM, BM), :]
    p = jnp.dot(a, g, preferred_element_type=jnp.float32)

    @pl.when(j == 0)
    def _init():
        acc[...] = p

    @pl.when(j > 0)
    def _accum():
        acc[...] += p

    @pl.when(j == nj - 1)
    def _finish():
        h_ref[0] = _leaky(acc[...])


def kernel(x, adj, W0, b0, W1, b1, W2, b2):
    B, Fi, N, T = x.shape
    C = T * F_OUT

    xt = x.transpose(0, 2, 3, 1).reshape(B, N * T, Fi)
    Wc = jnp.concatenate([W0, W1, W2], axis=1)
    bc = jnp.concatenate([b0, b1, b2]).reshape(1, 3 * F_OUT)

    y0, y1, y2 = pl.pallas_call(
        _prep_kernel,
        grid=(B, (N * T) // BR),
        in_specs=[
            pl.BlockSpec((1, BR, Fi), lambda b, i: (b, i, 0)),
            pl.BlockSpec((Fi, 3 * F_OUT), lambda b, i: (0, 0)),
            pl.BlockSpec((1, 3 * F_OUT), lambda b, i: (0, 0)),
        ],
        out_specs=[
            pl.BlockSpec((1, BR, F_OUT), lambda b, i: (b, i, 0)),
            pl.BlockSpec((1, BR, F_OUT), lambda b, i: (b, i, 0)),
            pl.BlockSpec((1, BR, F_OUT), lambda b, i: (b, i, 0)),
        ],
        out_shape=[jax.ShapeDtypeStruct((B, N * T, F_OUT), jnp.float32)] * 3,
    )(xt, Wc, bc)

    g1 = y1.reshape(B, N, C)
    g2 = y2.reshape(B, N, C)

    h1, u2 = pl.pallas_call(
        _spmm2_kernel,
        grid=(B, N // BN, N // BM),
        in_specs=[
            pl.BlockSpec((1, BN, BM), lambda b, i, j: (b, i, j)),
            pl.BlockSpec((1, N, C), lambda b, i, j: (b, 0, 0)),
            pl.BlockSpec((1, N, C), lambda b, i, j: (b, 0, 0)),
        ],
        out_specs=[
            pl.BlockSpec((1, BN, C), lambda b, i, j: (b, i, 0)),
            pl.BlockSpec((1, BN, C), lambda b, i, j: (b, i, 0)),
        ],
        out_shape=[jax.ShapeDtypeStruct((B, N, C), jnp.float32)] * 2,
        scratch_shapes=[
            pltpu.VMEM((BN, C), jnp.float32),
            pltpu.VMEM((BN, C), jnp.float32),
        ],
    )(adj, g1, g2)

    h2 = pl.pallas_call(
        _spmm1_kernel,
        grid=(B, N // BN, N // BM),
        in_specs=[
            pl.BlockSpec((1, BN, BM), lambda b, i, j: (b, i, j)),
            pl.BlockSpec((1, N, C), lambda b, i, j: (b, 0, 0)),
        ],
        out_specs=pl.BlockSpec((1, BN, C), lambda b, i, j: (b, i, 0)),
        out_shape=jax.ShapeDtypeStruct((B, N, C), jnp.float32),
        scratch_shapes=[pltpu.VMEM((BN, C), jnp.float32)],
    )(adj, u2)

    o0 = y0.reshape(B, N, T, F_OUT)
    o1 = h1.reshape(B, N, T, F_OUT)
    o2 = h2.reshape(B, N, T, F_OUT)
    return jnp.concatenate([o0, o1, o2], axis=-1).transpose(0, 3, 1, 2)


# speedup vs baseline: 2.0582x; 2.0582x over previous
import jax
import jax.numpy as jnp
from jax.experimental import pallas as pl

F_IN = 64
F_OUT = 32
NEG_SLOPE = 0.01

BN = 512
BP = 1024


def _leaky(v):
    return jnp.where(v >= 0, v, NEG_SLOPE * v)


def _prep_kernel(xt_ref, w_ref, b_ref, y0_ref, g_ref):
    y = jnp.dot(xt_ref[0], w_ref[...], preferred_element_type=jnp.float32)
    y = y + b_ref[0][None, :]
    C = y.shape[1] // 3
    y0_ref[0] = _leaky(y[:, :C])
    g_ref[0] = y[:, C:]


def _hop1_kernel(adj_ref, g_ref, h1_ref, u2_ref):
    u = jnp.dot(adj_ref[0], g_ref[0], preferred_element_type=jnp.float32)
    C = u.shape[1] // 2
    h1_ref[0] = _leaky(u[:, :C])
    u2_ref[0] = u[:, C:]


def _hop2_kernel(adj_ref, g_ref, h_ref):
    h_ref[0] = _leaky(
        jnp.dot(adj_ref[0], g_ref[0], preferred_element_type=jnp.float32))


def kernel(x, adj, W0, b0, W1, b1, W2, b2):
    B, Fi, N, T = x.shape
    C = T * F_OUT

    xt = x.transpose(0, 2, 3, 1).reshape(B, N, T * Fi)
    eyeT = jnp.eye(T, dtype=jnp.float32)
    Wc = jnp.concatenate(
        [jnp.kron(eyeT, W) for W in (W0, W1, W2)], axis=1)
    bc = jnp.concatenate(
        [jnp.tile(b, T) for b in (b0, b1, b2)]).reshape(1, 3 * C)

    y0, g = pl.pallas_call(
        _prep_kernel,
        grid=(B, N // BP),
        in_specs=[
            pl.BlockSpec((1, BP, T * Fi), lambda b, i: (b, i, 0)),
            pl.BlockSpec((T * Fi, 3 * C), lambda b, i: (0, 0)),
            pl.BlockSpec((1, 3 * C), lambda b, i: (0, 0)),
        ],
        out_specs=[
            pl.BlockSpec((1, BP, C), lambda b, i: (b, i, 0)),
            pl.BlockSpec((1, BP, 2 * C), lambda b, i: (b, i, 0)),
        ],
        out_shape=[
            jax.ShapeDtypeStruct((B, N, C), jnp.float32),
            jax.ShapeDtypeStruct((B, N, 2 * C), jnp.float32),
        ],
    )(xt, Wc, bc)

    h1, u2 = pl.pallas_call(
        _hop1_kernel,
        grid=(B, N // BN),
        in_specs=[
            pl.BlockSpec((1, BN, N), lambda b, i: (b, i, 0)),
            pl.BlockSpec((1, N, 2 * C), lambda b, i: (b, 0, 0)),
        ],
        out_specs=[
            pl.BlockSpec((1, BN, C), lambda b, i: (b, i, 0)),
            pl.BlockSpec((1, BN, C), lambda b, i: (b, i, 0)),
        ],
        out_shape=[jax.ShapeDtypeStruct((B, N, C), jnp.float32)] * 2,
    )(adj, g)

    h2 = pl.pallas_call(
        _hop2_kernel,
        grid=(B, N // BN),
        in_specs=[
            pl.BlockSpec((1, BN, N), lambda b, i: (b, i, 0)),
            pl.BlockSpec((1, N, C), lambda b, i: (b, 0, 0)),
        ],
        out_specs=pl.BlockSpec((1, BN, C), lambda b, i: (b, i, 0)),
        out_shape=jax.ShapeDtypeStruct((B, N, C), jnp.float32),
    )(adj, u2)

    o0 = y0.reshape(B, N, T, F_OUT)
    o1 = h1.reshape(B, N, T, F_OUT)
    o2 = h2.reshape(B, N, T, F_OUT)
    return jnp.concatenate([o0, o1, o2], axis=-1).transpose(0, 3, 1, 2)


# speedup vs baseline: 2.0938x; 1.0173x over previous
import jax
import jax.numpy as jnp
from jax.experimental import pallas as pl

F_IN = 64
F_OUT = 32
NEG_SLOPE = 0.01

BN = 512
BP = 1024


def _leaky(v):
    return jnp.where(v >= 0, v, NEG_SLOPE * v)


def _prep_kernel(xt_ref, w_ref, b_ref, y0_ref, g_ref):
    y = jnp.dot(xt_ref[0], w_ref[...], preferred_element_type=jnp.float32)
    y = y + b_ref[0][None, :]
    C = y.shape[1] // 3
    y0_ref[0] = _leaky(y[:, :C])
    g_ref[0] = y[:, C:].astype(jnp.bfloat16)


def _hop1_kernel(adj_ref, g_ref, h1_ref, u2_ref):
    a = adj_ref[0].astype(jnp.bfloat16)
    u = jnp.dot(a, g_ref[0], preferred_element_type=jnp.float32)
    C = u.shape[1] // 2
    h1_ref[0] = _leaky(u[:, :C])
    u2_ref[0] = u[:, C:].astype(jnp.bfloat16)


def _hop2_kernel(adj_ref, g_ref, h_ref):
    a = adj_ref[0].astype(jnp.bfloat16)
    h_ref[0] = _leaky(
        jnp.dot(a, g_ref[0], preferred_element_type=jnp.float32))


def kernel(x, adj, W0, b0, W1, b1, W2, b2):
    B, Fi, N, T = x.shape
    C = T * F_OUT

    xt = x.transpose(0, 2, 3, 1).reshape(B, N, T * Fi)
    eyeT = jnp.eye(T, dtype=jnp.float32)
    Wc = jnp.concatenate(
        [jnp.kron(eyeT, W) for W in (W0, W1, W2)], axis=1)
    bc = jnp.concatenate(
        [jnp.tile(b, T) for b in (b0, b1, b2)]).reshape(1, 3 * C)

    y0, g = pl.pallas_call(
        _prep_kernel,
        grid=(B, N // BP),
        in_specs=[
            pl.BlockSpec((1, BP, T * Fi), lambda b, i: (b, i, 0)),
            pl.BlockSpec((T * Fi, 3 * C), lambda b, i: (0, 0)),
            pl.BlockSpec((1, 3 * C), lambda b, i: (0, 0)),
        ],
        out_specs=[
            pl.BlockSpec((1, BP, C), lambda b, i: (b, i, 0)),
            pl.BlockSpec((1, BP, 2 * C), lambda b, i: (b, i, 0)),
        ],
        out_shape=[
            jax.ShapeDtypeStruct((B, N, C), jnp.float32),
            jax.ShapeDtypeStruct((B, N, 2 * C), jnp.bfloat16),
        ],
    )(xt, Wc, bc)

    h1, u2 = pl.pallas_call(
        _hop1_kernel,
        grid=(B, N // BN),
        in_specs=[
            pl.BlockSpec((1, BN, N), lambda b, i: (b, i, 0)),
            pl.BlockSpec((1, N, 2 * C), lambda b, i: (b, 0, 0)),
        ],
        out_specs=[
            pl.BlockSpec((1, BN, C), lambda b, i: (b, i, 0)),
            pl.BlockSpec((1, BN, C), lambda b, i: (b, i, 0)),
        ],
        out_shape=[
            jax.ShapeDtypeStruct((B, N, C), jnp.float32),
            jax.ShapeDtypeStruct((B, N, C), jnp.bfloat16),
        ],
    )(adj, g)

    h2 = pl.pallas_call(
        _hop2_kernel,
        grid=(B, N // BN),
        in_specs=[
            pl.BlockSpec((1, BN, N), lambda b, i: (b, i, 0)),
            pl.BlockSpec((1, N, C), lambda b, i: (b, 0, 0)),
        ],
        out_specs=pl.BlockSpec((1, BN, C), lambda b, i: (b, i, 0)),
        out_shape=jax.ShapeDtypeStruct((B, N, C), jnp.float32),
    )(adj, u2)

    o0 = y0.reshape(B, N, T, F_OUT)
    o1 = h1.reshape(B, N, T, F_OUT)
    o2 = h2.reshape(B, N, T, F_OUT)
    return jnp.concatenate([o0, o1, o2], axis=-1).transpose(0, 3, 1, 2)
